# Initial kernel scaffold; baseline (speedup 1.0000x reference)
import jax
import jax.numpy as jnp
from jax.experimental import pallas as pl


def kernel(x, var_constr_index, constr_var_index, W_init_var, b_init_var, W_init_con, b_init_con, W_var, b_var, W_con, b_con, W_q, b_q):
    raise NotImplementedError("write your pallas kernel here")


# speedup vs baseline: 1.9519x; 1.9519x over previous
import functools

import jax
import jax.numpy as jnp
from jax import lax
from jax.experimental import pallas as pl
from jax.experimental.pallas import tpu as pltpu
from jax.experimental.pallas import tpu_sc as plsc

NV = 25000
NC_NODES = 25000
D = 128
DEG = 16
NW = 32
PAD_N = 25088
ROWS_PER_W = PAD_N // NW
BLK = 8
NBLK = ROWS_PER_W // BLK
ROW_BLOCK = 1000
GRID = NV // ROW_BLOCK


def _gather_sum_body(table_hbm, idx_hbm, out_hbm, idx_v, buf_v, out_v, sem):
    wid = lax.axis_index("s") * 2 + lax.axis_index("c")
    base = wid * ROWS_PER_W

    def blk(b, carry):
        row0 = base + b * BLK
        pltpu.sync_copy(idx_hbm.at[pl.ds(row0 * DEG, BLK * DEG)], idx_v)
        pltpu.async_copy(table_hbm.at[idx_v], buf_v, sem).wait()
        for r in range(BLK):
            for g in range(D // 16):
                acc = buf_v[r * DEG, pl.ds(g * 16, 16)]
                for j in range(1, DEG):
                    acc = acc + buf_v[r * DEG + j, pl.ds(g * 16, 16)]
                out_v[r, pl.ds(g * 16, 16)] = acc
        pltpu.sync_copy(out_v, out_hbm.at[pl.ds(row0, BLK)])
        return carry

    lax.fori_loop(0, NBLK, blk, 0)


@functools.partial(jax.jit)
def _gather_sum(table, idx_flat):
    mesh = plsc.VectorSubcoreMesh(core_axis_name="c", subcore_axis_name="s")
    f = pl.kernel(
        _gather_sum_body,
        mesh=mesh,
        out_type=jax.ShapeDtypeStruct((PAD_N, D), jnp.float32),
        scratch_types=[
            pltpu.VMEM((BLK * DEG,), jnp.int32),
            pltpu.VMEM((BLK * DEG, D), jnp.float32),
            pltpu.VMEM((BLK, D), jnp.float32),
            pltpu.SemaphoreType.DMA,
        ],
    )
    return f(table, idx_flat)


def _dotT(a, w):
    return lax.dot_general(a, w, (((1,), (1,)), ((), ())),
                           preferred_element_type=jnp.float32)


def _init_body(xv_ref, xc_ref, wv_ref, bv_ref, wc_ref, bc_ref, v_ref, c_ref):
    v_ref[...] = _dotT(xv_ref[...], wv_ref[...]) + bv_ref[...]
    c_ref[...] = _dotT(xc_ref[...], wc_ref[...]) + bc_ref[...]


def _init_linears(xv, xc, wv, bv, wc, bc):
    row = pl.BlockSpec((ROW_BLOCK, D), lambda i: (i, 0))
    full = pl.BlockSpec((D, D), lambda i: (0, 0))
    bias = pl.BlockSpec((1, D), lambda i: (0, 0))
    return pl.pallas_call(
        _init_body,
        grid=(GRID,),
        in_specs=[row, row, full, bias, full, bias],
        out_specs=[row, row],
        out_shape=[jax.ShapeDtypeStruct((NV, D), jnp.float32),
                   jax.ShapeDtypeStruct((NC_NODES, D), jnp.float32)],
    )(xv, xc, wv, bv.reshape(1, D), wc, bc.reshape(1, D))


def _round_body(agg_ref, prev_ref, xb_ref, w_ref, b_ref, y_ref):
    w = w_ref[...]
    y = _dotT(agg_ref[...], w[:, 0:D])
    y += _dotT(prev_ref[...], w[:, D:2 * D])
    y += _dotT(xb_ref[...], w[:, 2 * D:3 * D])
    y_ref[...] = y + b_ref[...]


def _round_linear(agg, prev, xb, w, b):
    row = pl.BlockSpec((ROW_BLOCK, D), lambda i: (i, 0))
    wspec = pl.BlockSpec((D, 3 * D), lambda i: (0, 0))
    bias = pl.BlockSpec((1, D), lambda i: (0, 0))
    return pl.pallas_call(
        _round_body,
        grid=(GRID,),
        in_specs=[row, row, row, wspec, bias],
        out_specs=row,
        out_shape=jax.ShapeDtypeStruct((NV, D), jnp.float32),
    )(agg, prev, xb, w, b.reshape(1, D))


def _final_v_body(agg_ref, prev_ref, xb_ref, w_ref, b_ref, w2_ref,
                  rowdot_ref, colsum_ref):
    w = w_ref[...]
    y = _dotT(agg_ref[...], w[:, 0:D])
    y += _dotT(prev_ref[...], w[:, D:2 * D])
    y += _dotT(xb_ref[...], w[:, 2 * D:3 * D])
    y = y + b_ref[...]
    rowdot_ref[...] = _dotT(y, w2_ref[...])
    i = pl.program_id(0)

    @pl.when(i == 0)
    def _():
        colsum_ref[...] = jnp.zeros_like(colsum_ref)

    colsum_ref[...] += jnp.sum(y, axis=0, keepdims=True)


def _final_v(agg, prev, xb, w, b, w2):
    row = pl.BlockSpec((ROW_BLOCK, D), lambda i: (i, 0))
    wspec = pl.BlockSpec((D, 3 * D), lambda i: (0, 0))
    bias = pl.BlockSpec((1, D), lambda i: (0, 0))
    return pl.pallas_call(
        _final_v_body,
        grid=(GRID,),
        in_specs=[row, row, row, wspec, bias, bias],
        out_specs=[pl.BlockSpec((ROW_BLOCK, 1), lambda i: (i, 0)),
                   pl.BlockSpec((1, D), lambda i: (0, 0))],
        out_shape=[jax.ShapeDtypeStruct((NV, 1), jnp.float32),
                   jax.ShapeDtypeStruct((1, D), jnp.float32)],
    )(agg, prev, xb, w, b.reshape(1, D), w2)


def _q_body(rowdot_ref, colsum_ref, w1_ref, bq_ref, xcol_ref, q_ref):
    s = jnp.sum(colsum_ref[...] * w1_ref[...]) + bq_ref[0, 0]
    q = rowdot_ref[...] + s
    mask = xcol_ref[...].astype(jnp.int32) != 0
    q_ref[...] = jnp.where(mask, jnp.inf, q)


def _q_stage(rowdot, colsum, w1, bq, xcol):
    row1 = pl.BlockSpec((ROW_BLOCK, 1), lambda i: (i, 0))
    return pl.pallas_call(
        _q_body,
        grid=(GRID,),
        in_specs=[row1,
                  pl.BlockSpec((1, D), lambda i: (0, 0)),
                  pl.BlockSpec((1, D), lambda i: (0, 0)),
                  pl.BlockSpec((1, 1), lambda i: (0, 0)),
                  row1],
        out_specs=row1,
        out_shape=jax.ShapeDtypeStruct((NV, 1), jnp.float32),
    )(rowdot, colsum, w1, bq.reshape(1, 1), xcol)


def kernel(x, var_constr_index, constr_var_index, W_init_var, b_init_var,
           W_init_con, b_init_con, W_var, b_var, W_con, b_con, W_q, b_q):
    xv = x[:NV]
    xc = x[NV:]
    xcol = x[:NV, 1:2]

    pad = PAD_N - NV
    vci = jnp.pad(var_constr_index, ((0, pad), (0, 0))).reshape(-1)
    cvi = jnp.pad(constr_var_index, ((0, pad), (0, 0))).reshape(-1)

    last_v, last_c = _init_linears(xv, xc, W_init_var, b_init_var,
                                   W_init_con, b_init_con)

    agg_c = _gather_sum(last_v, cvi)[:NC_NODES]
    agg_v = _gather_sum(last_c, vci)[:NV]
    new_c = _round_linear(agg_c, last_c, xc, W_con, b_con)
    new_v = _round_linear(agg_v, last_v, xv, W_var, b_var)

    agg_v2 = _gather_sum(new_c, vci)[:NV]

    w1 = W_q[:, :D]
    w2 = W_q[:, D:]
    rowdot, colsum = _final_v(agg_v2, new_v, xv, W_var, b_var, w2)
    return _q_stage(rowdot, colsum, w1, b_q, xcol)


# speedup vs baseline: 2.8448x; 1.4574x over previous
import functools

import jax
import jax.numpy as jnp
from jax import lax
from jax.experimental import pallas as pl
from jax.experimental.pallas import tpu as pltpu
from jax.experimental.pallas import tpu_sc as plsc

NV = 25000
NC_NODES = 25000
D = 128
DEG = 16
NW = 32
PAD_N = 25088
ROWS_PER_W = PAD_N // NW
BLK = 8
NBLK = ROWS_PER_W // BLK
ROW_BLOCK = 1000
GRID = NV // ROW_BLOCK


def _reduce_block(buf_v, out_v, out_row0):
    for r in range(BLK):
        for g in range(D // 16):
            acc = buf_v[r * DEG, pl.ds(g * 16, 16)]
            for j in range(1, DEG):
                acc = acc + buf_v[r * DEG + j, pl.ds(g * 16, 16)]
            out_v[out_row0 + r, pl.ds(g * 16, 16)] = acc


def _gather_sum_body(table_hbm, idx_hbm, out_hbm, idx_v, buf0, buf1, out_v,
                     g0, g1, so):
    wid = lax.axis_index("s") * 2 + lax.axis_index("c")
    base = wid * ROWS_PER_W

    pltpu.sync_copy(idx_hbm.at[wid], idx_v)
    pltpu.async_copy(table_hbm.at[idx_v.at[0]], buf0, g0)

    npairs = NBLK // 2

    def pair(p, carry):
        b0 = p * 2
        pltpu.async_copy(table_hbm.at[idx_v.at[b0 + 1]], buf1, g1)
        pltpu.make_async_copy(table_hbm.at[idx_v.at[b0]], buf0, g0).wait()

        @pl.when(p > 0)
        def _():
            pltpu.make_async_copy(out_v, out_hbm.at[pl.ds(base, 2 * BLK)],
                                  so).wait()

        _reduce_block(buf0, out_v, 0)

        @pl.when(p < npairs - 1)
        def _():
            pltpu.async_copy(table_hbm.at[idx_v.at[b0 + 2]], buf0, g0)

        pltpu.make_async_copy(table_hbm.at[idx_v.at[b0 + 1]], buf1, g1).wait()
        _reduce_block(buf1, out_v, BLK)

        pltpu.async_copy(out_v, out_hbm.at[pl.ds(base + b0 * BLK, 2 * BLK)],
                         so)
        return carry

    lax.fori_loop(0, npairs, pair, 0)
    pltpu.make_async_copy(out_v, out_hbm.at[pl.ds(base, 2 * BLK)], so).wait()


@functools.partial(jax.jit)
def _gather_sum(table, idx_pages):
    mesh = plsc.VectorSubcoreMesh(core_axis_name="c", subcore_axis_name="s")
    f = pl.kernel(
        _gather_sum_body,
        mesh=mesh,
        out_type=jax.ShapeDtypeStruct((PAD_N, D), jnp.float32),
        scratch_types=[
            pltpu.VMEM((NBLK, BLK * DEG), jnp.int32),
            pltpu.VMEM((BLK * DEG, D), jnp.float32),
            pltpu.VMEM((BLK * DEG, D), jnp.float32),
            pltpu.VMEM((2 * BLK, D), jnp.float32),
            pltpu.SemaphoreType.DMA,
            pltpu.SemaphoreType.DMA,
            pltpu.SemaphoreType.DMA,
        ],
    )
    return f(table, idx_pages)


def _dotT(a, w):
    return lax.dot_general(a, w, (((1,), (1,)), ((), ())),
                           preferred_element_type=jnp.float32)


def _init_body(xv_ref, xc_ref, wv_ref, bv_ref, wc_ref, bc_ref, v_ref, c_ref):
    v_ref[...] = _dotT(xv_ref[...], wv_ref[...]) + bv_ref[...]
    c_ref[...] = _dotT(xc_ref[...], wc_ref[...]) + bc_ref[...]


def _init_linears(xv, xc, wv, bv, wc, bc):
    row = pl.BlockSpec((ROW_BLOCK, D), lambda i: (i, 0))
    full = pl.BlockSpec((D, D), lambda i: (0, 0))
    bias = pl.BlockSpec((1, D), lambda i: (0, 0))
    return pl.pallas_call(
        _init_body,
        grid=(GRID,),
        in_specs=[row, row, full, bias, full, bias],
        out_specs=[row, row],
        out_shape=[jax.ShapeDtypeStruct((NV, D), jnp.float32),
                   jax.ShapeDtypeStruct((NC_NODES, D), jnp.float32)],
    )(xv, xc, wv, bv.reshape(1, D), wc, bc.reshape(1, D))


def _round_body(agg_ref, prev_ref, xb_ref, w_ref, b_ref, y_ref):
    w = w_ref[...]
    y = _dotT(agg_ref[...], w[:, 0:D])
    y += _dotT(prev_ref[...], w[:, D:2 * D])
    y += _dotT(xb_ref[...], w[:, 2 * D:3 * D])
    y_ref[...] = y + b_ref[...]


def _round_linear(agg, prev, xb, w, b):
    row = pl.BlockSpec((ROW_BLOCK, D), lambda i: (i, 0))
    wspec = pl.BlockSpec((D, 3 * D), lambda i: (0, 0))
    bias = pl.BlockSpec((1, D), lambda i: (0, 0))
    return pl.pallas_call(
        _round_body,
        grid=(GRID,),
        in_specs=[row, row, row, wspec, bias],
        out_specs=row,
        out_shape=jax.ShapeDtypeStruct((NV, D), jnp.float32),
    )(agg, prev, xb, w, b.reshape(1, D))


def _final_v_body(agg_ref, prev_ref, xb_ref, w_ref, b_ref, w2_ref,
                  rowdot_ref, colsum_ref):
    w = w_ref[...]
    y = _dotT(agg_ref[...], w[:, 0:D])
    y += _dotT(prev_ref[...], w[:, D:2 * D])
    y += _dotT(xb_ref[...], w[:, 2 * D:3 * D])
    y = y + b_ref[...]
    rowdot_ref[...] = _dotT(y, w2_ref[...])
    i = pl.program_id(0)

    @pl.when(i == 0)
    def _():
        colsum_ref[...] = jnp.zeros_like(colsum_ref)

    colsum_ref[...] += jnp.sum(y, axis=0, keepdims=True)


def _final_v(agg, prev, xb, w, b, w2):
    row = pl.BlockSpec((ROW_BLOCK, D), lambda i: (i, 0))
    wspec = pl.BlockSpec((D, 3 * D), lambda i: (0, 0))
    bias = pl.BlockSpec((1, D), lambda i: (0, 0))
    return pl.pallas_call(
        _final_v_body,
        grid=(GRID,),
        in_specs=[row, row, row, wspec, bias, bias],
        out_specs=[pl.BlockSpec((ROW_BLOCK, 1), lambda i: (i, 0)),
                   pl.BlockSpec((1, D), lambda i: (0, 0))],
        out_shape=[jax.ShapeDtypeStruct((NV, 1), jnp.float32),
                   jax.ShapeDtypeStruct((1, D), jnp.float32)],
    )(agg, prev, xb, w, b.reshape(1, D), w2)


def _q_body(rowdot_ref, colsum_ref, w1_ref, bq_ref, xcol_ref, q_ref):
    s = jnp.sum(colsum_ref[...] * w1_ref[...]) + bq_ref[0, 0]
    q = rowdot_ref[...] + s
    mask = xcol_ref[...].astype(jnp.int32) != 0
    q_ref[...] = jnp.where(mask, jnp.inf, q)


def _q_stage(rowdot, colsum, w1, bq, xcol):
    row1 = pl.BlockSpec((ROW_BLOCK, 1), lambda i: (i, 0))
    return pl.pallas_call(
        _q_body,
        grid=(GRID,),
        in_specs=[row1,
                  pl.BlockSpec((1, D), lambda i: (0, 0)),
                  pl.BlockSpec((1, D), lambda i: (0, 0)),
                  pl.BlockSpec((1, 1), lambda i: (0, 0)),
                  row1],
        out_specs=row1,
        out_shape=jax.ShapeDtypeStruct((NV, 1), jnp.float32),
    )(rowdot, colsum, w1, bq.reshape(1, 1), xcol)


def kernel(x, var_constr_index, constr_var_index, W_init_var, b_init_var,
           W_init_con, b_init_con, W_var, b_var, W_con, b_con, W_q, b_q):
    xv = x[:NV]
    xc = x[NV:]
    xcol = x[:NV, 1:2]

    pad = PAD_N - NV
    vci = jnp.pad(var_constr_index, ((0, pad), (0, 0))).reshape(
        NW, NBLK, BLK * DEG)
    cvi = jnp.pad(constr_var_index, ((0, pad), (0, 0))).reshape(
        NW, NBLK, BLK * DEG)

    last_v, last_c = _init_linears(xv, xc, W_init_var, b_init_var,
                                   W_init_con, b_init_con)

    agg_c = _gather_sum(last_v, cvi)[:NC_NODES]
    agg_v = _gather_sum(last_c, vci)[:NV]
    new_c = _round_linear(agg_c, last_c, xc, W_con, b_con)
    new_v = _round_linear(agg_v, last_v, xv, W_var, b_var)

    agg_v2 = _gather_sum(new_c, vci)[:NV]

    w1 = W_q[:, :D]
    w2 = W_q[:, D:]
    rowdot, colsum = _final_v(agg_v2, new_v, xv, W_var, b_var, w2)
    return _q_stage(rowdot, colsum, w1, b_q, xcol)
